# baseline (speedup 1.0000x reference)
import functools
import math

import jax
import jax.numpy as jnp
from jax import lax
from jax.experimental import pallas as pl
from jax.experimental.pallas import tpu as pltpu
from jax.experimental.pallas import tpu_sc as plsc

D_MODEL = 64
SCALE = math.sqrt(D_MODEL)
CHUNK = 128


def kernel(tokens, table):
    b, s = tokens.shape
    v, d = table.shape
    tok_phys = tokens.T.astype(jnp.int32)
    table2 = table.reshape(v // 2, 2 * d)

    info = plsc.get_sparse_core_info()
    num_workers = info.num_cores * info.num_subcores
    nchunks = b // CHUNK

    mesh = plsc.VectorSubcoreMesh(core_axis_name="c", subcore_axis_name="s")

    @functools.partial(
        pl.kernel,
        mesh=mesh,
        out_type=jax.ShapeDtypeStruct((s, d, b), jnp.float32),
        scratch_types=[
            pltpu.VMEM((b,), jnp.int32),
            pltpu.VMEM((nchunks, CHUNK), jnp.int32),
            pltpu.VMEM((2, CHUNK, 2 * d), jnp.float32),
            pltpu.VMEM((2, d, CHUNK), jnp.float32),
        ]
        + [pltpu.SemaphoreType.DMA] * 4,
        compiler_params=pltpu.CompilerParams(needs_layout_passes=False),
    )
    def emb_kernel(tok_hbm, table_hbm, out_hbm, idx_full, idx_half, rows, out_t, *sems):
        gsems = sems[:2]
        osems = sems[2:]
        wid = lax.axis_index("s") * info.num_cores + lax.axis_index("c")
        nslices = (s - wid + num_workers - 1) // num_workers
        lane = jnp.arange(16, dtype=jnp.int32)
        rot = [(lane + dd) & 15 for dd in range(16)]

        def gather_start(j, p):
            pltpu.async_copy(table_hbm.at[idx_half.at[j]], rows.at[p], gsems[p])

        def gather_wait(j, p):
            pltpu.make_async_copy(
                table_hbm.at[idx_half.at[j]], rows.at[p], gsems[p]
            ).wait()

        def write_start(sl, j, p):
            pltpu.async_copy(
                out_t.at[p], out_hbm.at[sl, :, pl.ds(j * CHUNK, CHUNK)], osems[p]
            )

        def write_wait(sl, j, p):
            pltpu.make_async_copy(
                out_t.at[p], out_hbm.at[sl, :, pl.ds(j * CHUNK, CHUNK)], osems[p]
            ).wait()

        def transpose_scale(j, p):
            @plsc.parallel_loop(0, CHUNK // 16)
            def _(rb):
                idxv = idx_full[pl.ds(j * CHUNK + rb * 16, 16)]
                off = (idxv & 1) << 6
                ridx = rb * 16 + lane
                for cb in range(d // 16):
                    colbase = off + (cb * 16)
                    out_blk = out_t.at[p, pl.ds(cb * 16, 16)]
                    for dd in range(16):
                        vals = plsc.load_gather(
                            rows.at[p], [ridx, colbase + rot[dd]]
                        )
                        plsc.store_scatter(out_blk, [rot[dd], ridx], vals * SCALE)

        def slice_body(k, carry):
            sl = wid + k * num_workers
            pltpu.sync_copy(tok_hbm.at[sl], idx_full)

            @plsc.parallel_loop(0, b // 16)
            def _(g):
                vals = idx_full[pl.ds(g * 16, 16)] >> 1
                idx_half[g >> 3, pl.ds((g & 7) * 16, 16)] = vals

            gather_start(0, 0)
            gather_start(1, 1)

            def round_body(j2, carry2):
                for p in range(2):
                    j = j2 * 2 + p
                    gather_wait(j, p)

                    @pl.when(j >= 2)
                    def _():
                        write_wait(sl, j - 2, p)

                    transpose_scale(j, p)
                    write_start(sl, j, p)

                    @pl.when(j + 2 < nchunks)
                    def _():
                        gather_start(j + 2, p)

                return carry2

            lax.fori_loop(0, nchunks // 2, round_body, 0)
            write_wait(sl, nchunks - 2, 0)
            write_wait(sl, nchunks - 1, 1)
            return carry

        lax.fori_loop(0, nslices, slice_body, 0)

    out_phys = emb_kernel(tok_phys, table2)
    return jnp.transpose(out_phys, (2, 0, 1))


# speedup vs baseline: 1.5046x; 1.5046x over previous
import functools
import math

import jax
import jax.numpy as jnp
from jax import lax
from jax.experimental import pallas as pl
from jax.experimental.pallas import tpu as pltpu
from jax.experimental.pallas import tpu_sc as plsc

D_MODEL = 64
SCALE = math.sqrt(D_MODEL)
CHUNK = 128


def kernel(tokens, table):
    b, s = tokens.shape
    v, d = table.shape
    tok_phys = tokens.T.astype(jnp.int32)
    table2 = table.reshape(v // 2, 2 * d)

    info = plsc.get_sparse_core_info()
    num_workers = info.num_cores * info.num_subcores
    nchunks = b // CHUNK

    mesh = plsc.VectorSubcoreMesh(core_axis_name="c", subcore_axis_name="s")

    @functools.partial(
        pl.kernel,
        mesh=mesh,
        out_type=jax.ShapeDtypeStruct((s, d, b), jnp.float32),
        scratch_types=[
            pltpu.VMEM((b,), jnp.int32),
            pltpu.VMEM((nchunks, CHUNK), jnp.int32),
            pltpu.VMEM((2, CHUNK, 2 * d), jnp.float32),
            pltpu.VMEM((2, d, CHUNK), jnp.float32),
        ]
        + [pltpu.SemaphoreType.DMA] * 4,
        compiler_params=pltpu.CompilerParams(needs_layout_passes=False),
    )
    def emb_kernel(tok_hbm, table_hbm, out_hbm, idx_full, idx_half, rows, out_t, *sems):
        gsems = sems[:2]
        osems = sems[2:]
        wid = lax.axis_index("s") * info.num_cores + lax.axis_index("c")
        nslices = (s - wid + num_workers - 1) // num_workers
        lane = jnp.arange(16, dtype=jnp.int32)
        rot = [(lane + dd) & 15 for dd in range(16)]

        def gather_start(j, p):
            pltpu.async_copy(table_hbm.at[idx_half.at[j]], rows.at[p], gsems[p])

        def gather_wait(j, p):
            pltpu.make_async_copy(
                table_hbm.at[idx_half.at[j]], rows.at[p], gsems[p]
            ).wait()

        def write_start(sl, j, p):
            pltpu.async_copy(
                out_t.at[p], out_hbm.at[sl, :, pl.ds(j * CHUNK, CHUNK)], osems[p]
            )

        def write_wait(sl, j, p):
            pltpu.make_async_copy(
                out_t.at[p], out_hbm.at[sl, :, pl.ds(j * CHUNK, CHUNK)], osems[p]
            ).wait()

        def transpose_scale(j, p):
            @plsc.parallel_loop(0, (CHUNK // 16) * 16, unroll=2)
            def _(t):
                rb = t >> 4
                dd = t & 15
                idxv = idx_full[pl.ds(j * CHUNK + rb * 16, 16)]
                off = (idxv & 1) << 6
                ridx = rb * 16 + lane
                rotv = (lane + dd) & 15
                colbase = off + rotv
                for cb in range(d // 16):
                    crot = cb * 16 + rotv
                    vals = plsc.load_gather(rows.at[p], [ridx, colbase + cb * 16])
                    plsc.store_scatter(out_t.at[p], [crot, ridx], vals * SCALE)

        def slice_body(k, carry):
            sl = wid + k * num_workers
            pltpu.sync_copy(tok_hbm.at[sl], idx_full)

            @plsc.parallel_loop(0, b // 16)
            def _(g):
                vals = idx_full[pl.ds(g * 16, 16)] >> 1
                idx_half[g >> 3, pl.ds((g & 7) * 16, 16)] = vals

            gather_start(0, 0)
            gather_start(1, 1)

            def round_body(j2, carry2):
                for p in range(2):
                    j = j2 * 2 + p
                    gather_wait(j, p)

                    @pl.when(j >= 2)
                    def _():
                        write_wait(sl, j - 2, p)

                    transpose_scale(j, p)
                    write_start(sl, j, p)

                    @pl.when(j + 2 < nchunks)
                    def _():
                        gather_start(j + 2, p)

                return carry2

            lax.fori_loop(0, nchunks // 2, round_body, 0)
            write_wait(sl, nchunks - 2, 0)
            write_wait(sl, nchunks - 1, 1)
            return carry

        lax.fori_loop(0, nslices, slice_body, 0)

    out_phys = emb_kernel(tok_phys, table2)
    return jnp.transpose(out_phys, (2, 0, 1))


# speedup vs baseline: 2.5653x; 1.7050x over previous
import functools
import math

import jax
import jax.numpy as jnp
from jax import lax
from jax.experimental import pallas as pl
from jax.experimental.pallas import tpu as pltpu
from jax.experimental.pallas import tpu_sc as plsc

D_MODEL = 64
SCALE = math.sqrt(D_MODEL)
CHUNK = 128


def kernel(tokens, table):
    b, s = tokens.shape
    v, d = table.shape
    tok_phys = tokens.T.astype(jnp.int32)
    table_t = table.T

    info = plsc.get_sparse_core_info()
    num_workers = info.num_cores * info.num_subcores
    nchunks = b // CHUNK

    mesh = plsc.VectorSubcoreMesh(core_axis_name="c", subcore_axis_name="s")

    nblk = v // 128
    tail = v - nblk * 128

    @functools.partial(
        pl.kernel,
        mesh=mesh,
        out_type=jax.ShapeDtypeStruct((v // 2, 2 * d), jnp.float32),
        scratch_types=[
            pltpu.VMEM((2, d, 128), jnp.float32),
            pltpu.VMEM((2, d, 2 * d), jnp.float32),
        ]
        + [pltpu.SemaphoreType.DMA] * 4,
        compiler_params=pltpu.CompilerParams(needs_layout_passes=False),
    )
    def relayout_kernel(tab_hbm, tail_hbm, out_hbm, in_b, out_b, *sems):
        isems = sems[:2]
        osems = sems[2:]
        wid = lax.axis_index("s") * info.num_cores + lax.axis_index("c")
        kmax = 2 * ((nblk // num_workers + 2) // 2)
        lane = jnp.arange(16, dtype=jnp.int32)

        def in_start(blk, p, w):
            pltpu.async_copy(
                tab_hbm.at[:, pl.ds(blk * 128, w)],
                in_b.at[p, :, pl.ds(0, w)],
                isems[p],
            )

        def in_wait(p, w):
            pltpu.make_async_copy(
                tab_hbm.at[:, pl.ds(0, w)], in_b.at[p, :, pl.ds(0, w)], isems[p]
            ).wait()

        def out_start(blk, p, w):
            pltpu.async_copy(
                out_b.at[p, pl.ds(0, w // 2)],
                out_hbm.at[pl.ds(blk * 64, w // 2)],
                osems[p],
            )

        def out_wait(p, w):
            pltpu.make_async_copy(
                out_b.at[p, pl.ds(0, w // 2)],
                out_hbm.at[pl.ds(0, w // 2)],
                osems[p],
            ).wait()

        def transpose_block(p, w):
            @plsc.parallel_loop(0, (w // 16) * 16, unroll=2)
            def _(t2):
                tb = t2 >> 4
                dd = t2 & 15
                rotv = (lane + dd) & 15
                tvec = tb * 16 + rotv
                rv = tvec >> 1
                cbase = ((tvec & 1) << 6) + lane
                for cb in range(d // 16):
                    vals = plsc.load_gather(in_b.at[p], [cb * 16 + lane, tvec])
                    plsc.store_scatter(out_b.at[p], [rv, cbase + cb * 16], vals)

        in_start(wid, 0, 128)
        in_start(wid + num_workers, 1, 128)

        def round_body(k2, carry):
            for p in range(2):
                k = k2 * 2 + p
                blk = wid + k * num_workers
                live = blk < nblk

                @pl.when(live & (k2 >= 1))
                def _():
                    out_wait(p, 128)

                @pl.when(live)
                def _():
                    in_wait(p, 128)
                    transpose_block(p, 128)

                @pl.when((blk + 2 * num_workers) < nblk)
                def _():
                    in_start(blk + 2 * num_workers, p, 128)

                @pl.when(live)
                def _():
                    out_start(blk, p, 128)

            return carry

        lax.fori_loop(0, kmax // 2, round_body, 0)
        out_wait(0, 128)
        out_wait(1, 128)

        if tail:

            @pl.when(wid == num_workers - 1)
            def _():
                tr = tail // 2
                pltpu.sync_copy(tail_hbm, in_b.at[0, pl.ds(0, tr)])
                pltpu.sync_copy(
                    in_b.at[0, pl.ds(0, tr)], out_hbm.at[pl.ds(nblk * 64, tr)]
                )

    @functools.partial(
        pl.kernel,
        mesh=mesh,
        out_type=jax.ShapeDtypeStruct((s, d, b), jnp.float32),
        scratch_types=[
            pltpu.VMEM((b,), jnp.int32),
            pltpu.VMEM((nchunks, CHUNK), jnp.int32),
            pltpu.VMEM((2, CHUNK, 2 * d), jnp.float32),
            pltpu.VMEM((2, d, CHUNK), jnp.float32),
        ]
        + [pltpu.SemaphoreType.DMA] * 4,
        compiler_params=pltpu.CompilerParams(needs_layout_passes=False),
    )
    def emb_kernel(tok_hbm, table_hbm, out_hbm, idx_full, idx_half, rows, out_t, *sems):
        gsems = sems[:2]
        osems = sems[2:]
        wid = lax.axis_index("s") * info.num_cores + lax.axis_index("c")
        nslices = (s - wid + num_workers - 1) // num_workers
        lane = jnp.arange(16, dtype=jnp.int32)
        rot = [(lane + dd) & 15 for dd in range(16)]

        def gather_start(j, p):
            pltpu.async_copy(table_hbm.at[idx_half.at[j]], rows.at[p], gsems[p])

        def gather_wait(j, p):
            pltpu.make_async_copy(
                table_hbm.at[idx_half.at[j]], rows.at[p], gsems[p]
            ).wait()

        def write_start(sl, j, p):
            pltpu.async_copy(
                out_t.at[p], out_hbm.at[sl, :, pl.ds(j * CHUNK, CHUNK)], osems[p]
            )

        def write_wait(sl, j, p):
            pltpu.make_async_copy(
                out_t.at[p], out_hbm.at[sl, :, pl.ds(j * CHUNK, CHUNK)], osems[p]
            ).wait()

        def transpose_scale(j, p):
            @plsc.parallel_loop(0, (CHUNK // 16) * 16, unroll=2)
            def _(t):
                rb = t >> 4
                dd = t & 15
                idxv = idx_full[pl.ds(j * CHUNK + rb * 16, 16)]
                off = (idxv & 1) << 6
                ridx = rb * 16 + lane
                rotv = (lane + dd) & 15
                colbase = off + rotv
                for cb in range(d // 16):
                    crot = cb * 16 + rotv
                    vals = plsc.load_gather(rows.at[p], [ridx, colbase + cb * 16])
                    plsc.store_scatter(out_t.at[p], [crot, ridx], vals * SCALE)

        def slice_body(k, carry):
            sl = wid + k * num_workers
            pltpu.sync_copy(tok_hbm.at[sl], idx_full)

            @plsc.parallel_loop(0, b // 16)
            def _(g):
                vals = idx_full[pl.ds(g * 16, 16)] >> 1
                idx_half[g >> 3, pl.ds((g & 7) * 16, 16)] = vals

            gather_start(0, 0)
            gather_start(1, 1)

            def round_body(j2, carry2):
                for p in range(2):
                    j = j2 * 2 + p
                    gather_wait(j, p)

                    @pl.when(j >= 2)
                    def _():
                        write_wait(sl, j - 2, p)

                    transpose_scale(j, p)
                    write_start(sl, j, p)

                    @pl.when(j + 2 < nchunks)
                    def _():
                        gather_start(j + 2, p)

                return carry2

            lax.fori_loop(0, nchunks // 2, round_body, 0)
            write_wait(sl, nchunks - 2, 0)
            write_wait(sl, nchunks - 1, 1)
            return carry

        lax.fori_loop(0, nslices, slice_body, 0)

    tail_rows = table[v - tail :, :].reshape(tail // 2, 2 * d)
    table2 = relayout_kernel(table_t, tail_rows)
    out_phys = emb_kernel(tok_phys, table2)
    return jnp.transpose(out_phys, (2, 0, 1))


# speedup vs baseline: 2.6016x; 1.0141x over previous
import functools
import math

import jax
import jax.numpy as jnp
from jax import lax
from jax.experimental import pallas as pl
from jax.experimental.pallas import tpu as pltpu
from jax.experimental.pallas import tpu_sc as plsc

D_MODEL = 64
SCALE = math.sqrt(D_MODEL)
CHUNK = 128


def kernel(tokens, table):
    b, s = tokens.shape
    v, d = table.shape
    tok_phys = tokens.T.astype(jnp.int32)
    table_t = table.T

    info = plsc.get_sparse_core_info()
    num_workers = info.num_cores * info.num_subcores
    nchunks = b // CHUNK

    mesh = plsc.VectorSubcoreMesh(core_axis_name="c", subcore_axis_name="s")

    nblk = v // 128
    tail = v - nblk * 128

    @functools.partial(
        pl.kernel,
        mesh=mesh,
        out_type=jax.ShapeDtypeStruct((v // 2, 2 * d), jnp.float32),
        scratch_types=[
            pltpu.VMEM((2, d, 128), jnp.float32),
            pltpu.VMEM((2, d, 2 * d), jnp.float32),
        ]
        + [pltpu.SemaphoreType.DMA] * 4,
        compiler_params=pltpu.CompilerParams(needs_layout_passes=False),
    )
    def relayout_kernel(tab_hbm, tail_hbm, out_hbm, in_b, out_b, *sems):
        isems = sems[:2]
        osems = sems[2:]
        wid = lax.axis_index("s") * info.num_cores + lax.axis_index("c")
        kmax = 2 * ((nblk // num_workers + 2) // 2)
        lane = jnp.arange(16, dtype=jnp.int32)

        def in_start(blk, p, w):
            pltpu.async_copy(
                tab_hbm.at[:, pl.ds(blk * 128, w)],
                in_b.at[p, :, pl.ds(0, w)],
                isems[p],
            )

        def in_wait(p, w):
            pltpu.make_async_copy(
                tab_hbm.at[:, pl.ds(0, w)], in_b.at[p, :, pl.ds(0, w)], isems[p]
            ).wait()

        def out_start(blk, p, w):
            pltpu.async_copy(
                out_b.at[p, pl.ds(0, w // 2)],
                out_hbm.at[pl.ds(blk * 64, w // 2)],
                osems[p],
            )

        def out_wait(p, w):
            pltpu.make_async_copy(
                out_b.at[p, pl.ds(0, w // 2)],
                out_hbm.at[pl.ds(0, w // 2)],
                osems[p],
            ).wait()

        def transpose_block(p, w):
            @plsc.parallel_loop(0, (w // 16) * 16, unroll=4)
            def _(t2):
                tb = t2 >> 4
                dd = t2 & 15
                rotv = (lane + dd) & 15
                tvec = tb * 16 + rotv
                rv = tvec >> 1
                cbase = ((tvec & 1) << 6) + lane
                for cb in range(d // 16):
                    vals = plsc.load_gather(in_b.at[p], [cb * 16 + lane, tvec])
                    plsc.store_scatter(out_b.at[p], [rv, cbase + cb * 16], vals)

        in_start(wid, 0, 128)
        in_start(wid + num_workers, 1, 128)

        def round_body(k2, carry):
            for p in range(2):
                k = k2 * 2 + p
                blk = wid + k * num_workers
                live = blk < nblk

                @pl.when(live & (k2 >= 1))
                def _():
                    out_wait(p, 128)

                @pl.when(live)
                def _():
                    in_wait(p, 128)
                    transpose_block(p, 128)

                @pl.when((blk + 2 * num_workers) < nblk)
                def _():
                    in_start(blk + 2 * num_workers, p, 128)

                @pl.when(live)
                def _():
                    out_start(blk, p, 128)

            return carry

        lax.fori_loop(0, kmax // 2, round_body, 0)
        out_wait(0, 128)
        out_wait(1, 128)

        if tail:

            @pl.when(wid == num_workers - 1)
            def _():
                tr = tail // 2
                pltpu.sync_copy(tail_hbm, in_b.at[0, pl.ds(0, tr)])
                pltpu.sync_copy(
                    in_b.at[0, pl.ds(0, tr)], out_hbm.at[pl.ds(nblk * 64, tr)]
                )

    @functools.partial(
        pl.kernel,
        mesh=mesh,
        out_type=jax.ShapeDtypeStruct((s, d, b), jnp.float32),
        scratch_types=[
            pltpu.VMEM((b,), jnp.int32),
            pltpu.VMEM((nchunks, CHUNK), jnp.int32),
            pltpu.VMEM((2, CHUNK, 2 * d), jnp.float32),
            pltpu.VMEM((2, d, CHUNK), jnp.float32),
        ]
        + [pltpu.SemaphoreType.DMA] * 4,
        compiler_params=pltpu.CompilerParams(needs_layout_passes=False),
    )
    def emb_kernel(tok_hbm, table_hbm, out_hbm, idx_full, idx_half, rows, out_t, *sems):
        gsems = sems[:2]
        osems = sems[2:]
        wid = lax.axis_index("s") * info.num_cores + lax.axis_index("c")
        nslices = (s - wid + num_workers - 1) // num_workers
        lane = jnp.arange(16, dtype=jnp.int32)
        rot = [(lane + dd) & 15 for dd in range(16)]

        def gather_start(j, p):
            pltpu.async_copy(table_hbm.at[idx_half.at[j]], rows.at[p], gsems[p])

        def gather_wait(j, p):
            pltpu.make_async_copy(
                table_hbm.at[idx_half.at[j]], rows.at[p], gsems[p]
            ).wait()

        def write_start(sl, j, p):
            pltpu.async_copy(
                out_t.at[p], out_hbm.at[sl, :, pl.ds(j * CHUNK, CHUNK)], osems[p]
            )

        def write_wait(sl, j, p):
            pltpu.make_async_copy(
                out_t.at[p], out_hbm.at[sl, :, pl.ds(j * CHUNK, CHUNK)], osems[p]
            ).wait()

        def transpose_scale(j, p):
            @plsc.parallel_loop(0, (CHUNK // 16) * 16, unroll=4)
            def _(t):
                rb = t >> 4
                dd = t & 15
                idxv = idx_full[pl.ds(j * CHUNK + rb * 16, 16)]
                off = (idxv & 1) << 6
                ridx = rb * 16 + lane
                rotv = (lane + dd) & 15
                colbase = off + rotv
                for cb in range(d // 16):
                    crot = cb * 16 + rotv
                    vals = plsc.load_gather(rows.at[p], [ridx, colbase + cb * 16])
                    plsc.store_scatter(out_t.at[p], [crot, ridx], vals * SCALE)

        def slice_body(k, carry):
            sl = wid + k * num_workers
            pltpu.sync_copy(tok_hbm.at[sl], idx_full)

            @plsc.parallel_loop(0, b // 16)
            def _(g):
                vals = idx_full[pl.ds(g * 16, 16)] >> 1
                idx_half[g >> 3, pl.ds((g & 7) * 16, 16)] = vals

            gather_start(0, 0)
            gather_start(1, 1)

            def round_body(j2, carry2):
                for p in range(2):
                    j = j2 * 2 + p
                    gather_wait(j, p)

                    @pl.when(j >= 2)
                    def _():
                        write_wait(sl, j - 2, p)

                    transpose_scale(j, p)
                    write_start(sl, j, p)

                    @pl.when(j + 2 < nchunks)
                    def _():
                        gather_start(j + 2, p)

                return carry2

            lax.fori_loop(0, nchunks // 2, round_body, 0)
            write_wait(sl, nchunks - 2, 0)
            write_wait(sl, nchunks - 1, 1)
            return carry

        lax.fori_loop(0, nslices, slice_body, 0)

    tail_rows = table[v - tail :, :].reshape(tail // 2, 2 * d)
    table2 = relayout_kernel(table_t, tail_rows)
    out_phys = emb_kernel(tok_phys, table2)
    return jnp.transpose(out_phys, (2, 0, 1))


# speedup vs baseline: 2.8120x; 1.0809x over previous
import functools
import math

import jax
import jax.numpy as jnp
from jax import lax
from jax.experimental import pallas as pl
from jax.experimental.pallas import tpu as pltpu
from jax.experimental.pallas import tpu_sc as plsc

D_MODEL = 64
SCALE = math.sqrt(D_MODEL)
CHUNK = 128


def kernel(tokens, table):
    b, s = tokens.shape
    v, d = table.shape
    tok_phys = tokens.T.astype(jnp.int32)
    table_t = table.T

    info = plsc.get_sparse_core_info()
    num_workers = info.num_cores * info.num_subcores
    nchunks = b // CHUNK

    mesh = plsc.VectorSubcoreMesh(core_axis_name="c", subcore_axis_name="s")

    nblk = v // 128
    tail = v - nblk * 128

    @functools.partial(
        pl.kernel,
        mesh=mesh,
        out_type=jax.ShapeDtypeStruct((v // 2, 2 * d), jnp.float32),
        scratch_types=[
            pltpu.VMEM((2, d, 128), jnp.float32),
            pltpu.VMEM((2, d, 2 * d), jnp.float32),
        ]
        + [pltpu.SemaphoreType.DMA] * 4,
        compiler_params=pltpu.CompilerParams(needs_layout_passes=False),
    )
    def relayout_kernel(tab_hbm, tail_hbm, out_hbm, in_b, out_b, *sems):
        isems = sems[:2]
        osems = sems[2:]
        wid = lax.axis_index("s") * info.num_cores + lax.axis_index("c")
        kmax = 2 * ((nblk // num_workers + 2) // 2)
        lane = jnp.arange(16, dtype=jnp.int32)

        def in_start(blk, p, w):
            pltpu.async_copy(
                tab_hbm.at[:, pl.ds(blk * 128, w)],
                in_b.at[p, :, pl.ds(0, w)],
                isems[p],
            )

        def in_wait(p, w):
            pltpu.make_async_copy(
                tab_hbm.at[:, pl.ds(0, w)], in_b.at[p, :, pl.ds(0, w)], isems[p]
            ).wait()

        def out_start(blk, p, w):
            pltpu.async_copy(
                out_b.at[p, pl.ds(0, w // 2)],
                out_hbm.at[pl.ds(blk * 64, w // 2)],
                osems[p],
            )

        def out_wait(p, w):
            pltpu.make_async_copy(
                out_b.at[p, pl.ds(0, w // 2)],
                out_hbm.at[pl.ds(0, w // 2)],
                osems[p],
            ).wait()

        def transpose_block(p, w):
            @plsc.parallel_loop(0, (w // 16) * 16, unroll=4)
            def _(t2):
                tb = t2 >> 4
                dd = t2 & 15
                rotv = (lane + dd) & 15
                tvec = tb * 16 + rotv
                rv = tvec >> 1
                cbase = ((tvec & 1) << 6) + lane
                for cb in range(d // 16):
                    vals = plsc.load_gather(in_b.at[p], [cb * 16 + lane, tvec])
                    plsc.store_scatter(out_b.at[p], [rv, cbase + cb * 16], vals)

        in_start(wid, 0, 128)
        in_start(wid + num_workers, 1, 128)

        def round_body(k2, carry):
            for p in range(2):
                k = k2 * 2 + p
                blk = wid + k * num_workers
                live = blk < nblk

                @pl.when(live & (k2 >= 1))
                def _():
                    out_wait(p, 128)

                @pl.when(live)
                def _():
                    in_wait(p, 128)
                    transpose_block(p, 128)

                @pl.when((blk + 2 * num_workers) < nblk)
                def _():
                    in_start(blk + 2 * num_workers, p, 128)

                @pl.when(live)
                def _():
                    out_start(blk, p, 128)

            return carry

        lax.fori_loop(0, kmax // 2, round_body, 0)
        out_wait(0, 128)
        out_wait(1, 128)

        if tail:

            @pl.when(wid == num_workers - 1)
            def _():
                tr = tail // 2
                pltpu.sync_copy(tail_hbm, in_b.at[0, pl.ds(0, tr)])
                pltpu.sync_copy(
                    in_b.at[0, pl.ds(0, tr)], out_hbm.at[pl.ds(nblk * 64, tr)]
                )

    @functools.partial(
        pl.kernel,
        mesh=mesh,
        out_type=jax.ShapeDtypeStruct((s, d, b), jnp.float32),
        scratch_types=[
            pltpu.VMEM((s, CHUNK), jnp.int32),
            pltpu.VMEM((s, CHUNK), jnp.int32),
            pltpu.VMEM((2, CHUNK, 2 * d), jnp.float32),
            pltpu.VMEM((2, d, CHUNK), jnp.float32),
        ]
        + [pltpu.SemaphoreType.DMA] * 4,
        compiler_params=pltpu.CompilerParams(needs_layout_passes=False),
    )
    def emb_kernel(tok_hbm, table_hbm, out_hbm, idx_full, idx_half, rows, out_t, *sems):
        gsems = sems[:2]
        osems = sems[2:]
        wid = lax.axis_index("s") * info.num_cores + lax.axis_index("c")
        col0 = wid * CHUNK
        lane = jnp.arange(16, dtype=jnp.int32)

        pltpu.sync_copy(tok_hbm.at[:, pl.ds(col0, CHUNK)], idx_full)

        @plsc.parallel_loop(0, (s * CHUNK) // 16, unroll=4)
        def _(g):
            vals = idx_full[g >> 3, pl.ds((g & 7) * 16, 16)] >> 1
            idx_half[g >> 3, pl.ds((g & 7) * 16, 16)] = vals

        def gather_start(k, p):
            pltpu.async_copy(table_hbm.at[idx_half.at[k]], rows.at[p], gsems[p])

        def gather_wait(k, p):
            pltpu.make_async_copy(
                table_hbm.at[idx_half.at[k]], rows.at[p], gsems[p]
            ).wait()

        def write_start(k, p):
            pltpu.async_copy(
                out_t.at[p], out_hbm.at[k, :, pl.ds(col0, CHUNK)], osems[p]
            )

        def write_wait(k, p):
            pltpu.make_async_copy(
                out_t.at[p], out_hbm.at[k, :, pl.ds(col0, CHUNK)], osems[p]
            ).wait()

        def transpose_scale(k, p):
            @plsc.parallel_loop(0, (CHUNK // 16) * 16, unroll=4)
            def _(t):
                rb = t >> 4
                dd = t & 15
                idxv = idx_full[k, pl.ds(rb * 16, 16)]
                off = (idxv & 1) << 6
                ridx = rb * 16 + lane
                rotv = (lane + dd) & 15
                colbase = off + rotv
                for cb in range(d // 16):
                    crot = cb * 16 + rotv
                    vals = plsc.load_gather(rows.at[p], [ridx, colbase + cb * 16])
                    plsc.store_scatter(out_t.at[p], [crot, ridx], vals * SCALE)

        gather_start(0, 0)
        gather_start(1, 1)

        def round_body(k2, carry2):
            for p in range(2):
                k = k2 * 2 + p
                gather_wait(k, p)

                @pl.when(k >= 2)
                def _():
                    write_wait(k - 2, p)

                transpose_scale(k, p)
                write_start(k, p)

                @pl.when(k + 2 < s)
                def _():
                    gather_start(k + 2, p)

            return carry2

        lax.fori_loop(0, s // 2, round_body, 0)
        write_wait(s - 2, 0)
        write_wait(s - 1, 1)

    tail_rows = table[v - tail :, :].reshape(tail // 2, 2 * d)
    table2 = relayout_kernel(table_t, tail_rows)
    out_phys = emb_kernel(tok_phys, table2)
    return jnp.transpose(out_phys, (2, 0, 1))


# speedup vs baseline: 3.0491x; 1.0843x over previous
import functools
import math

import jax
import jax.numpy as jnp
from jax import lax
from jax.experimental import pallas as pl
from jax.experimental.pallas import tpu as pltpu
from jax.experimental.pallas import tpu_sc as plsc

D_MODEL = 64
SCALE = math.sqrt(D_MODEL)
CHUNK = 128


def kernel(tokens, table):
    b, s = tokens.shape
    v, d = table.shape
    tok_phys = tokens.T.astype(jnp.int32)
    table_t = table.T

    info = plsc.get_sparse_core_info()
    num_workers = info.num_cores * info.num_subcores
    nchunks = b // CHUNK

    mesh = plsc.VectorSubcoreMesh(core_axis_name="c", subcore_axis_name="s")

    BLKW = 256
    nblk = v // BLKW
    tail = v - nblk * BLKW

    @functools.partial(
        pl.kernel,
        mesh=mesh,
        out_type=jax.ShapeDtypeStruct((v // 2, 2 * d), jnp.float32),
        scratch_types=[
            pltpu.VMEM((2, d, BLKW), jnp.float32),
            pltpu.VMEM((2, BLKW // 2, 2 * d), jnp.float32),
        ]
        + [pltpu.SemaphoreType.DMA] * 4,
        compiler_params=pltpu.CompilerParams(needs_layout_passes=False),
    )
    def relayout_kernel(tab_hbm, tail_hbm, out_hbm, in_b, out_b, *sems):
        isems = sems[:2]
        osems = sems[2:]
        wid = lax.axis_index("s") * info.num_cores + lax.axis_index("c")
        kmax = 2 * ((nblk // num_workers + 2) // 2)
        lane = jnp.arange(16, dtype=jnp.int32)

        def in_start(blk, p, w):
            pltpu.async_copy(
                tab_hbm.at[:, pl.ds(blk * BLKW, w)],
                in_b.at[p, :, pl.ds(0, w)],
                isems[p],
            )

        def in_wait(p, w):
            pltpu.make_async_copy(
                tab_hbm.at[:, pl.ds(0, w)], in_b.at[p, :, pl.ds(0, w)], isems[p]
            ).wait()

        def out_start(blk, p, w):
            pltpu.async_copy(
                out_b.at[p, pl.ds(0, w // 2)],
                out_hbm.at[pl.ds(blk * (BLKW // 2), w // 2)],
                osems[p],
            )

        def out_wait(p, w):
            pltpu.make_async_copy(
                out_b.at[p, pl.ds(0, w // 2)],
                out_hbm.at[pl.ds(0, w // 2)],
                osems[p],
            ).wait()

        def transpose_block(p, w):
            @plsc.parallel_loop(0, (w // 16) * 16, unroll=4)
            def _(t2):
                tb = t2 >> 4
                dd = t2 & 15
                rotv = (lane + dd) & 15
                tvec = tb * 16 + rotv
                rv = tvec >> 1
                cbase = ((tvec & 1) << 6) + lane
                for cb in range(d // 16):
                    vals = plsc.load_gather(in_b.at[p], [cb * 16 + lane, tvec])
                    plsc.store_scatter(out_b.at[p], [rv, cbase + cb * 16], vals)

        in_start(wid, 0, BLKW)
        in_start(wid + num_workers, 1, BLKW)

        def round_body(k2, carry):
            for p in range(2):
                k = k2 * 2 + p
                blk = wid + k * num_workers
                live = blk < nblk

                @pl.when(live & (k2 >= 1))
                def _():
                    out_wait(p, BLKW)

                @pl.when(live)
                def _():
                    in_wait(p, BLKW)
                    transpose_block(p, BLKW)

                @pl.when((blk + 2 * num_workers) < nblk)
                def _():
                    in_start(blk + 2 * num_workers, p, BLKW)

                @pl.when(live)
                def _():
                    out_start(blk, p, BLKW)

            return carry

        lax.fori_loop(0, kmax // 2, round_body, 0)
        out_wait(0, BLKW)
        out_wait(1, BLKW)

        if tail:

            @pl.when(wid == num_workers - 1)
            def _():
                tr = tail // 2
                stage = in_b.at[0, pl.ds(0, tr), pl.ds(0, 2 * d)]
                pltpu.sync_copy(tail_hbm, stage)
                pltpu.sync_copy(stage, out_hbm.at[pl.ds(nblk * (BLKW // 2), tr)])

    @functools.partial(
        pl.kernel,
        mesh=mesh,
        out_type=jax.ShapeDtypeStruct((s, d, b), jnp.float32),
        scratch_types=[
            pltpu.VMEM((s, CHUNK), jnp.int32),
            pltpu.VMEM((s, CHUNK), jnp.int32),
            pltpu.VMEM((2, CHUNK, 2 * d), jnp.float32),
            pltpu.VMEM((2, d, CHUNK), jnp.float32),
        ]
        + [pltpu.SemaphoreType.DMA] * 4,
        compiler_params=pltpu.CompilerParams(needs_layout_passes=False),
    )
    def emb_kernel(tok_hbm, table_hbm, out_hbm, idx_full, idx_half, rows, out_t, *sems):
        gsems = sems[:2]
        osems = sems[2:]
        wid = lax.axis_index("s") * info.num_cores + lax.axis_index("c")
        col0 = wid * CHUNK
        lane = jnp.arange(16, dtype=jnp.int32)

        pltpu.sync_copy(tok_hbm.at[:, pl.ds(col0, CHUNK)], idx_full)

        @plsc.parallel_loop(0, (s * CHUNK) // 16, unroll=4)
        def _(g):
            vals = idx_full[g >> 3, pl.ds((g & 7) * 16, 16)] >> 1
            idx_half[g >> 3, pl.ds((g & 7) * 16, 16)] = vals

        def gather_start(k, p):
            pltpu.async_copy(table_hbm.at[idx_half.at[k]], rows.at[p], gsems[p])

        def gather_wait(k, p):
            pltpu.make_async_copy(
                table_hbm.at[idx_half.at[k]], rows.at[p], gsems[p]
            ).wait()

        def write_start(k, p):
            pltpu.async_copy(
                out_t.at[p], out_hbm.at[k, :, pl.ds(col0, CHUNK)], osems[p]
            )

        def write_wait(k, p):
            pltpu.make_async_copy(
                out_t.at[p], out_hbm.at[k, :, pl.ds(col0, CHUNK)], osems[p]
            ).wait()

        def transpose_scale(k, p):
            @plsc.parallel_loop(0, (CHUNK // 16) * 16, unroll=4)
            def _(t):
                rb = t >> 4
                dd = t & 15
                idxv = idx_full[k, pl.ds(rb * 16, 16)]
                off = (idxv & 1) << 6
                ridx = rb * 16 + lane
                rotv = (lane + dd) & 15
                colbase = off + rotv
                for cb in range(d // 16):
                    crot = cb * 16 + rotv
                    vals = plsc.load_gather(rows.at[p], [ridx, colbase + cb * 16])
                    plsc.store_scatter(out_t.at[p], [crot, ridx], vals * SCALE)

        gather_start(0, 0)
        gather_start(1, 1)

        def round_body(k2, carry2):
            for p in range(2):
                k = k2 * 2 + p
                gather_wait(k, p)

                @pl.when(k >= 2)
                def _():
                    write_wait(k - 2, p)

                transpose_scale(k, p)
                write_start(k, p)

                @pl.when(k + 2 < s)
                def _():
                    gather_start(k + 2, p)

            return carry2

        lax.fori_loop(0, s // 2, round_body, 0)
        write_wait(s - 2, 0)
        write_wait(s - 1, 1)

    tail_rows = table[v - tail :, :].reshape(tail // 2, 2 * d)
    table2 = relayout_kernel(table_t, tail_rows)
    out_phys = emb_kernel(tok_phys, table2)
    return jnp.transpose(out_phys, (2, 0, 1))
